# baseline (speedup 1.0000x reference)
import jax
import jax.numpy as jnp
from jax import lax
from jax.experimental import pallas as pl
from jax.experimental.pallas import tpu as pltpu
from jax.experimental.pallas import tpu_sc as plsc

H = W = 384
C = 192
R = 16
NW = 32
RPW = H // NW
BH = 64
NG = H // BH


def _mesh():
    return plsc.VectorSubcoreMesh(core_axis_name="c", subcore_axis_name="s",
                                  num_cores=2, num_subcores=16)


def _sload(ref, i):
    return ref[pl.ds(i, 16)][0]


def _k1_body(rowid_smem, xt_ref, mcol_ref, mcolt_ref, recip_ref, out_ref,
             a_scr, rs_scr):
    g = pl.program_id(0)

    @pl.when(g == 0)
    def _():
        a_scr[...] = jnp.zeros_like(a_scr)

    for h in range(BH):
        rs_scr[h] = jnp.dot(xt_ref[0, h], mcol_ref[...],
                            preferred_element_type=jnp.float32,
                            precision=lax.Precision.HIGHEST)

    def body(h, carry):
        r = rowid_smem[g * BH + h]
        a_scr[r] = a_scr[r] + rs_scr[h]
        return carry

    lax.fori_loop(0, BH, body, 0)

    @pl.when(g == NG - 1)
    def _():
        for r in range(R):
            m = a_scr[r] * recip_ref[r].reshape(1, R)
            out_ref[r] = jnp.dot(m, mcolt_ref[...],
                                 preferred_element_type=jnp.float32,
                                 precision=lax.Precision.HIGHEST)


def _expanded_rows_tc(xt, mcol, mcolt, recip, row_id):
    return pl.pallas_call(
        _k1_body,
        grid_spec=pltpu.PrefetchScalarGridSpec(
            num_scalar_prefetch=1,
            grid=(NG,),
            in_specs=[
                pl.BlockSpec((1, BH, C, W), lambda g, s: (0, g, 0, 0)),
                pl.BlockSpec((W, R), lambda g, s: (0, 0)),
                pl.BlockSpec((R, W), lambda g, s: (0, 0)),
                pl.BlockSpec((R, R), lambda g, s: (0, 0)),
            ],
            out_specs=pl.BlockSpec((R, C, W), lambda g, s: (0, 0, 0)),
            scratch_shapes=[pltpu.VMEM((R, C, R), jnp.float32),
                            pltpu.VMEM((BH, C, R), jnp.float32)],
        ),
        out_shape=jax.ShapeDtypeStruct((R, C, W), jnp.float32),
    )(row_id, xt, mcol, mcolt, recip)


def _k2_body(mexp_hbm, rowid_hbm, out_hbm, rowbuf, rowid_s):
    cid = lax.axis_index("c")
    sid = lax.axis_index("s")
    wid = sid * 2 + cid
    base = wid * RPW

    pltpu.sync_copy(rowid_hbm, rowid_s.at[pl.ds(0, H)])

    @pl.loop(0, RPW, init_carry=jnp.int32(-1))
    def _(i, r_prev):
        h = base + i
        r = _sload(rowid_s, h)

        @pl.when(r != r_prev)
        def _():
            pltpu.sync_copy(mexp_hbm.at[r], rowbuf)

        pltpu.sync_copy(rowbuf, out_hbm.at[0, h])
        return r


def _broadcast_sc(mexp, row_id):
    k2 = pl.kernel(
        _k2_body,
        out_type=jax.ShapeDtypeStruct((1, H, C, W), jnp.float32),
        mesh=_mesh(),
        scratch_types=[
            pltpu.VMEM((C, W), jnp.float32),
            pltpu.VMEM((H + 16,), jnp.int32),
        ],
    )
    return k2(mexp, row_id)


def kernel(input, h_positions, v_positions):
    hp = h_positions.astype(jnp.int32)
    vp = v_positions.astype(jnp.int32)
    ys = jnp.arange(H, dtype=jnp.int32)
    row_id = (ys[:, None] >= hp[None, :]).sum(axis=1, dtype=jnp.int32)
    col_id = (ys[:, None] >= vp[None, :]).sum(axis=1, dtype=jnp.int32)
    segs = jnp.arange(R, dtype=jnp.int32)
    mcol = (col_id[:, None] == segs[None, :]).astype(jnp.float32)
    mcolt = (col_id[None, :] == segs[:, None]).astype(jnp.float32)

    zero = jnp.zeros((1,), jnp.int32)
    h_bounds = jnp.concatenate([zero, hp, jnp.full((1,), H, jnp.int32)])
    v_bounds = jnp.concatenate([zero, vp, jnp.full((1,), W, jnp.int32)])
    row_h = (h_bounds[1:] - h_bounds[:-1]).astype(jnp.float32)
    col_w = (v_bounds[1:] - v_bounds[:-1]).astype(jnp.float32)
    area = row_h[:, None] * col_w[None, :]
    recip = 1.0 / jnp.maximum(area, 1.0)

    xt = jnp.transpose(input, (0, 1, 3, 2))
    mexp = _expanded_rows_tc(xt, mcol, mcolt, recip, row_id)
    out_t = _broadcast_sc(mexp, row_id)
    return jnp.transpose(out_t, (0, 1, 3, 2))
